# Initial kernel scaffold; baseline (speedup 1.0000x reference)
import jax
import jax.numpy as jnp
from jax import lax
from jax.experimental import pallas as pl
from jax.experimental.pallas import tpu as pltpu
from jax.experimental.pallas import tpu_sc as plsc

VOCAB = 100000
D = 128
W = 200
B = 1024

NC, NS = 2, 16
NW = NC * NS
ROWS_PER_W = B // NW
H = 2
WH = W // H


def _body(tab_hbm, tok_hbm, pos_hbm, out_hbm, idx_v, pos_v, buf):
    wid = lax.axis_index("s") * NC + lax.axis_index("c")
    pltpu.sync_copy(pos_hbm, pos_v)
    for r in range(ROWS_PER_W):
        row = wid * ROWS_PER_W + r
        pltpu.sync_copy(tok_hbm.at[row], idx_v)
        pltpu.sync_copy(pos_v, buf)
        for h in range(H):
            pltpu.sync_copy(
                tab_hbm.at[idx_v.at[h]],
                buf.at[pl.ds(h * WH, WH)],
                add=True,
            )
        pltpu.sync_copy(buf, out_hbm.at[pl.ds(row * W, W)])


def kernel(tokens, token_embedding, position_embedding, position_indices):
    del position_indices
    tokens3 = tokens.reshape(B, H, WH).astype(jnp.int32)
    mesh = plsc.VectorSubcoreMesh(
        core_axis_name="c", subcore_axis_name="s",
        num_cores=NC, num_subcores=NS,
    )
    out = pl.kernel(
        _body,
        out_type=jax.ShapeDtypeStruct((B * W, D), jnp.float32),
        mesh=mesh,
        scratch_types=[
            pltpu.VMEM((H, WH), jnp.int32),
            pltpu.VMEM((W, D), jnp.float32),
            pltpu.VMEM((W, D), jnp.float32),
        ],
    )(token_embedding, tokens3, position_embedding)
    return out.reshape(B, W, D)


# speedup vs baseline: 3.9801x; 3.9801x over previous
import jax
import jax.numpy as jnp
from jax import lax
from jax.experimental import pallas as pl
from jax.experimental.pallas import tpu as pltpu
from jax.experimental.pallas import tpu_sc as plsc

VOCAB = 100000
D = 128
W = 200
B = 1024

NC, NS = 2, 16
NW = NC * NS
ROWS_PER_W = B // NW
H = 2
WH = W // H


def _body(tab_hbm, tok_hbm, pos_hbm, out_hbm, idx_v, pos_v, buf):
    sid = lax.axis_index("s")
    wid = sid * NC + lax.axis_index("c")

    @pl.when(sid == 0)
    def _load_pos():
        pltpu.sync_copy(pos_hbm, pos_v)

    plsc.subcore_barrier()
    for r in range(ROWS_PER_W):
        row = wid * ROWS_PER_W + r
        pltpu.sync_copy(tok_hbm.at[row], idx_v)
        pltpu.sync_copy(pos_v, buf)
        for h in range(H):
            pltpu.sync_copy(
                tab_hbm.at[idx_v.at[h]],
                buf.at[pl.ds(h * WH, WH)],
                add=True,
            )
        pltpu.sync_copy(buf, out_hbm.at[pl.ds(row * W, W)])


def kernel(tokens, token_embedding, position_embedding, position_indices):
    del position_indices
    tokens3 = tokens.reshape(B, H, WH).astype(jnp.int32)
    mesh = plsc.VectorSubcoreMesh(
        core_axis_name="c", subcore_axis_name="s",
        num_cores=NC, num_subcores=NS,
    )
    out = pl.kernel(
        _body,
        out_type=jax.ShapeDtypeStruct((B * W, D), jnp.float32),
        mesh=mesh,
        scratch_types=[
            pltpu.VMEM((H, WH), jnp.int32),
            pltpu.VMEM_SHARED((W, D), jnp.float32),
            pltpu.VMEM((W, D), jnp.float32),
        ],
    )(token_embedding, tokens3, position_embedding)
    return out.reshape(B, W, D)


# speedup vs baseline: 6.0949x; 1.5313x over previous
import jax
import jax.numpy as jnp
from jax import lax
from jax.experimental import pallas as pl
from jax.experimental.pallas import tpu as pltpu
from jax.experimental.pallas import tpu_sc as plsc

VOCAB = 100000
D = 128
W = 200
B = 1024

NC, NS = 2, 16
NW = NC * NS
ROWS_PER_W = B // NW
H = 2
WH = W // H


def _body(tab_hbm, tok_hbm, pos_hbm, out_hbm,
          idx_v, pos_s, buf_a, buf_b,
          sem_idx, sem_pos_a, sem_pos_b, sem_gat_a, sem_gat_b,
          sem_out_a, sem_out_b):
    sid = lax.axis_index("s")
    wid = sid * NC + lax.axis_index("c")

    @pl.when(sid == 0)
    def _load_pos():
        pltpu.sync_copy(pos_hbm, pos_s)

    d_idx = pltpu.async_copy(tok_hbm.at[pl.ds(wid * ROWS_PER_W, ROWS_PER_W)],
                             idx_v, sem_idx)
    plsc.subcore_barrier()

    bufs = [buf_a, buf_b]
    sem_pos = [sem_pos_a, sem_pos_b]
    sem_gat = [sem_gat_a, sem_gat_b]
    sem_out = [sem_out_a, sem_out_b]
    d_pos = [None, None]
    d_out = [None, None]

    d_pos[0] = pltpu.async_copy(pos_s, bufs[0], sem_pos[0])
    d_idx.wait()

    for j in range(ROWS_PER_W):
        s = j % 2
        s2 = (j + 1) % 2
        if j + 1 < ROWS_PER_W:
            if d_out[s2] is not None:
                d_out[s2].wait()
                d_out[s2] = None
            d_pos[s2] = pltpu.async_copy(pos_s, bufs[s2], sem_pos[s2])
        d_pos[s].wait()
        d_g = []
        for h in range(H):
            d_g.append(pltpu.async_copy(
                tab_hbm.at[idx_v.at[j].at[h]],
                bufs[s].at[pl.ds(h * WH, WH)],
                sem_gat[s], add=True))
        for d in d_g:
            d.wait()
        row = wid * ROWS_PER_W + j
        d_out[s] = pltpu.async_copy(bufs[s], out_hbm.at[pl.ds(row * W, W)],
                                    sem_out[s])

    for s in range(2):
        if d_out[s] is not None:
            d_out[s].wait()


def kernel(tokens, token_embedding, position_embedding, position_indices):
    del position_indices
    tokens3 = tokens.reshape(B, H, WH).astype(jnp.int32)
    mesh = plsc.VectorSubcoreMesh(
        core_axis_name="c", subcore_axis_name="s",
        num_cores=NC, num_subcores=NS,
    )
    out = pl.kernel(
        _body,
        out_type=jax.ShapeDtypeStruct((B * W, D), jnp.float32),
        mesh=mesh,
        scratch_types=[
            pltpu.VMEM((ROWS_PER_W, H, WH), jnp.int32),
            pltpu.VMEM_SHARED((W, D), jnp.float32),
            pltpu.VMEM((W, D), jnp.float32),
            pltpu.VMEM((W, D), jnp.float32),
            pltpu.SemaphoreType.DMA,
            pltpu.SemaphoreType.DMA,
            pltpu.SemaphoreType.DMA,
            pltpu.SemaphoreType.DMA,
            pltpu.SemaphoreType.DMA,
            pltpu.SemaphoreType.DMA,
            pltpu.SemaphoreType.DMA,
        ],
    )(token_embedding, tokens3, position_embedding)
    return out.reshape(B, W, D)


# speedup vs baseline: 7.5515x; 1.2390x over previous
import jax
import jax.numpy as jnp
from jax import lax
from jax.experimental import pallas as pl
from jax.experimental.pallas import tpu as pltpu
from jax.experimental.pallas import tpu_sc as plsc

VOCAB = 100000
D = 128
W = 200
B = 1024

NC, NS = 2, 16
NW = NC * NS
ROWS_PER_W = B // NW
H = 2
WH = W // H


NBUF = 3


def _body(tab_hbm, tok_hbm, pos_hbm, out_hbm,
          idx_v, pos_s, buf_a, buf_b, buf_c,
          sem_idx, sem_pos_a, sem_pos_b, sem_pos_c,
          sem_gat_a, sem_gat_b, sem_gat_c,
          sem_out_a, sem_out_b, sem_out_c):
    sid = lax.axis_index("s")
    wid = sid * NC + lax.axis_index("c")

    @pl.when(sid == 0)
    def _load_pos():
        pltpu.sync_copy(pos_hbm, pos_s)

    d_idx = pltpu.async_copy(tok_hbm.at[pl.ds(wid * ROWS_PER_W, ROWS_PER_W)],
                             idx_v, sem_idx)
    plsc.subcore_barrier()

    bufs = [buf_a, buf_b, buf_c]
    sem_pos = [sem_pos_a, sem_pos_b, sem_pos_c]
    sem_gat = [sem_gat_a, sem_gat_b, sem_gat_c]
    sem_out = [sem_out_a, sem_out_b, sem_out_c]
    d_pos = [None] * NBUF
    d_out = [None] * NBUF
    d_gat = [None] * NBUF

    def start_pos(jw):
        s = jw % NBUF
        if d_out[s] is not None:
            d_out[s].wait()
            d_out[s] = None
        d_pos[s] = pltpu.async_copy(pos_s, bufs[s], sem_pos[s])

    def start_gather(jw):
        s = jw % NBUF
        d_pos[s].wait()
        d_gat[s] = [
            pltpu.async_copy(
                tab_hbm.at[idx_v.at[jw].at[h]],
                bufs[s].at[pl.ds(h * WH, WH)],
                sem_gat[s], add=True)
            for h in range(H)
        ]

    def finish(jw):
        s = jw % NBUF
        for d in d_gat[s]:
            d.wait()
        row = wid * ROWS_PER_W + jw
        d_out[s] = pltpu.async_copy(bufs[s], out_hbm.at[pl.ds(row * W, W)],
                                    sem_out[s])

    for jw in range(NBUF - 1):
        d_pos[jw] = pltpu.async_copy(pos_s, bufs[jw], sem_pos[jw])
    d_idx.wait()
    start_gather(0)

    for j in range(ROWS_PER_W):
        if j + 1 < ROWS_PER_W:
            start_gather(j + 1)
        if j + 2 < ROWS_PER_W:
            start_pos(j + 2)
        finish(j)

    for s in range(NBUF):
        if d_out[s] is not None:
            d_out[s].wait()


def kernel(tokens, token_embedding, position_embedding, position_indices):
    del position_indices
    tokens3 = tokens.reshape(B, H, WH).astype(jnp.int32)
    mesh = plsc.VectorSubcoreMesh(
        core_axis_name="c", subcore_axis_name="s",
        num_cores=NC, num_subcores=NS,
    )
    out = pl.kernel(
        _body,
        out_type=jax.ShapeDtypeStruct((B * W, D), jnp.float32),
        mesh=mesh,
        scratch_types=[
            pltpu.VMEM((ROWS_PER_W, H, WH), jnp.int32),
            pltpu.VMEM_SHARED((W, D), jnp.float32),
            pltpu.VMEM((W, D), jnp.float32),
            pltpu.VMEM((W, D), jnp.float32),
            pltpu.VMEM((W, D), jnp.float32),
        ] + [pltpu.SemaphoreType.DMA] * 10,
    )(token_embedding, tokens3, position_embedding)
    return out.reshape(B, W, D)


# speedup vs baseline: 7.6038x; 1.0069x over previous
import jax
import jax.numpy as jnp
from jax import lax
from jax.experimental import pallas as pl
from jax.experimental.pallas import tpu as pltpu
from jax.experimental.pallas import tpu_sc as plsc

VOCAB = 100000
D = 128
W = 200
B = 1024

NC, NS = 2, 16
NW = NC * NS
ROWS_PER_W = B // NW
H = 2
WH = W // H
NBUF = 4


def _body(tab_hbm, tok_hbm, pos_hbm, out_hbm, idx_v, pos_s, *scratch):
    bufs = list(scratch[:NBUF])
    sem_idx = scratch[NBUF]
    sem_pos = list(scratch[NBUF + 1:NBUF + 1 + NBUF])
    sem_gat = list(scratch[NBUF + 1 + NBUF:NBUF + 1 + 2 * NBUF])
    sem_out = list(scratch[NBUF + 1 + 2 * NBUF:NBUF + 1 + 3 * NBUF])

    sid = lax.axis_index("s")
    wid = sid * NC + lax.axis_index("c")

    @pl.when(sid == 0)
    def _load_pos():
        pltpu.sync_copy(pos_hbm, pos_s)

    d_idx = pltpu.async_copy(tok_hbm.at[pl.ds(wid * ROWS_PER_W, ROWS_PER_W)],
                             idx_v, sem_idx)
    plsc.subcore_barrier()

    d_pos = [None] * NBUF
    d_out = [None] * NBUF
    d_gat = [None] * NBUF

    def start_pos(jw):
        s = jw % NBUF
        if d_out[s] is not None:
            d_out[s].wait()
            d_out[s] = None
        d_pos[s] = pltpu.async_copy(pos_s, bufs[s], sem_pos[s])

    def start_gather(jw):
        s = jw % NBUF
        d_pos[s].wait()
        d_gat[s] = [
            pltpu.async_copy(
                tab_hbm.at[idx_v.at[jw].at[h]],
                bufs[s].at[pl.ds(h * WH, WH)],
                sem_gat[s], add=True)
            for h in range(H)
        ]

    def finish(jw):
        s = jw % NBUF
        for d in d_gat[s]:
            d.wait()
        row = wid * ROWS_PER_W + jw
        d_out[s] = pltpu.async_copy(bufs[s], out_hbm.at[pl.ds(row * W, W)],
                                    sem_out[s])

    for jw in range(NBUF - 1):
        d_pos[jw] = pltpu.async_copy(pos_s, bufs[jw], sem_pos[jw])
    d_idx.wait()
    start_gather(0)

    for j in range(ROWS_PER_W):
        if j + 1 < ROWS_PER_W:
            start_gather(j + 1)
        if j + NBUF - 1 < ROWS_PER_W:
            start_pos(j + NBUF - 1)
        finish(j)

    for s in range(NBUF):
        if d_out[s] is not None:
            d_out[s].wait()


def kernel(tokens, token_embedding, position_embedding, position_indices):
    del position_indices
    tokens3 = tokens.reshape(B, H, WH).astype(jnp.int32)
    mesh = plsc.VectorSubcoreMesh(
        core_axis_name="c", subcore_axis_name="s",
        num_cores=NC, num_subcores=NS,
    )
    out = pl.kernel(
        _body,
        out_type=jax.ShapeDtypeStruct((B * W, D), jnp.float32),
        mesh=mesh,
        scratch_types=[
            pltpu.VMEM((ROWS_PER_W, H, WH), jnp.int32),
            pltpu.VMEM_SHARED((W, D), jnp.float32),
        ] + [pltpu.VMEM((W, D), jnp.float32)] * NBUF
          + [pltpu.SemaphoreType.DMA] * (1 + 3 * NBUF),
    )(token_embedding, tokens3, position_embedding)
    return out.reshape(B, W, D)
